# Initial kernel scaffold; baseline (speedup 1.0000x reference)
import functools

import jax
import jax.numpy as jnp
from jax import lax
from jax.experimental import pallas as pl
from jax.experimental.pallas import tpu as pltpu
from jax.experimental.pallas import tpu_sc as plsc

N = 10000
E = 320000
D = 128

NC = 2
NS = 16
NW = NC * NS
EPW = E // NW
C = 80
NCH = EPW // C
RPS = N // NS

_MESH = plsc.VectorSubcoreMesh(
    core_axis_name="c", subcore_axis_name="s", num_cores=NC, num_subcores=NS
)


@functools.partial(
    pl.kernel,
    out_type=jax.ShapeDtypeStruct((NC, N), jnp.float32),
    mesh=_MESH,
    scratch_types=[
        pltpu.VMEM((NCH, C), jnp.int32),
        pltpu.VMEM((NCH, C), jnp.float32),
        pltpu.VMEM_SHARED((N,), jnp.float32),
    ],
)
def _deg(dst_hbm, ew_hbm, zn_hbm, out_hbm, dstv, ewv, degsh):
    ci = lax.axis_index("c")
    si = lax.axis_index("s")
    wid = ci * NS + si
    pltpu.sync_copy(dst_hbm.at[wid], dstv)
    pltpu.sync_copy(ew_hbm.at[wid], ewv)

    @pl.when(si == 0)
    def _():
        pltpu.sync_copy(zn_hbm, degsh)

    plsc.subcore_barrier()

    def body(j, carry):
        pltpu.sync_copy(ewv.at[j], degsh.at[dstv.at[j]], add=True)
        return carry

    lax.fori_loop(0, NCH, body, 0)
    plsc.subcore_barrier()

    @pl.when(si == 0)
    def _():
        pltpu.sync_copy(degsh, out_hbm.at[ci])


@functools.partial(
    pl.kernel,
    out_type=jax.ShapeDtypeStruct((NC, N, D), jnp.float32),
    mesh=_MESH,
    scratch_types=[
        pltpu.VMEM((NCH, C), jnp.int32),
        pltpu.VMEM((NCH, C), jnp.int32),
        pltpu.VMEM((C, NS), jnp.float32),
        pltpu.VMEM((C, D), jnp.float32),
        pltpu.VMEM_SHARED((N, D), jnp.float32),
        pltpu.SemaphoreType.DMA,
    ],
)
def _edge(h_hbm, src_hbm, dst_hbm, ew16_hbm, znd_hbm, out_hbm,
          srcv, dstv, ew16v, rows, acc, gsem):
    ci = lax.axis_index("c")
    si = lax.axis_index("s")
    wid = ci * NS + si
    pltpu.sync_copy(src_hbm.at[wid], srcv)
    pltpu.sync_copy(dst_hbm.at[wid], dstv)
    pltpu.sync_copy(znd_hbm.at[pl.ds(si * RPS, RPS)], acc.at[pl.ds(si * RPS, RPS)])
    plsc.subcore_barrier()

    def body(j, carry):
        pltpu.sync_copy(ew16_hbm.at[wid, j], ew16v)
        pltpu.async_copy(h_hbm.at[srcv.at[j]], rows, gsem).wait()
        for e in range(C):
            wv = ew16v[e]
            for q in range(D // 16):
                rows[e, 16 * q:16 * (q + 1)] = rows[e, 16 * q:16 * (q + 1)] * wv
        pltpu.sync_copy(rows, acc.at[dstv.at[j]], add=True)
        return carry

    lax.fori_loop(0, NCH, body, 0)
    plsc.subcore_barrier()
    pltpu.sync_copy(acc.at[pl.ds(si * RPS, RPS)],
                    out_hbm.at[ci, pl.ds(si * RPS, RPS)])


_RB = 2000
_GRID = N // _RB


def _dis_of(degT_blk):
    deg = degT_blk[:, 0:1] + degT_blk[:, 1:2] + 1.0
    return lax.rsqrt(deg)


def _tc1_body(x_ref, w1_ref, degT_ref, out_ref):
    dis = _dis_of(degT_ref[...])
    out_ref[...] = jnp.dot(x_ref[...], w1_ref[...],
                           preferred_element_type=jnp.float32) * dis


def _tc2_body(p_ref, h_ref, degT_ref, b1_ref, g_ref, be_ref, mu_ref, va_ref,
              w2_ref, out_ref):
    dis = _dis_of(degT_ref[...])
    t = dis * (p_ref[0] + p_ref[1] + h_ref[...]) + b1_ref[...]
    inv = lax.rsqrt(va_ref[...] + 1e-5)
    t = (t - mu_ref[...]) * inv * g_ref[...] + be_ref[...]
    t = jnp.maximum(t, 0.0)
    out_ref[...] = jnp.dot(t, w2_ref[...],
                           preferred_element_type=jnp.float32) * dis


def _tc3_body(p_ref, h_ref, degT_ref, b2_ref, out_ref):
    dis = _dis_of(degT_ref[...])
    out_ref[...] = dis * (p_ref[0] + p_ref[1] + h_ref[...]) + b2_ref[...]


_rowspec = pl.BlockSpec((_RB, D), lambda i: (i, 0))
_fullmat = pl.BlockSpec((D, D), lambda i: (0, 0))
_degspec = pl.BlockSpec((_RB, NC), lambda i: (i, 0))
_vecspec = pl.BlockSpec((1, D), lambda i: (0, 0))
_partspec = pl.BlockSpec((NC, _RB, D), lambda i: (0, i, 0))

_tc1 = pl.pallas_call(
    _tc1_body,
    grid=(_GRID,),
    in_specs=[_rowspec, _fullmat, _degspec],
    out_specs=_rowspec,
    out_shape=jax.ShapeDtypeStruct((N, D), jnp.float32),
)

_tc2 = pl.pallas_call(
    _tc2_body,
    grid=(_GRID,),
    in_specs=[_partspec, _rowspec, _degspec,
              _vecspec, _vecspec, _vecspec, _vecspec, _vecspec, _fullmat],
    out_specs=_rowspec,
    out_shape=jax.ShapeDtypeStruct((N, D), jnp.float32),
)

_tc3 = pl.pallas_call(
    _tc3_body,
    grid=(_GRID,),
    in_specs=[_partspec, _rowspec, _degspec, _vecspec],
    out_specs=_rowspec,
    out_shape=jax.ShapeDtypeStruct((N, D), jnp.float32),
)


def kernel(x, edge_index, edge_weight, W1, b1, bn_gamma, bn_beta, bn_mean,
           bn_var, W2, b2):
    src3 = edge_index[0].reshape(NW, NCH, C)
    dst3 = edge_index[1].reshape(NW, NCH, C)
    ew3 = edge_weight.reshape(NW, NCH, C)
    ew16 = jnp.broadcast_to(edge_weight[:, None], (E, NS)).reshape(NW, NCH, C, NS)
    zn = jnp.zeros((N,), jnp.float32)
    znd = jnp.zeros((N, D), jnp.float32)

    degP = _deg(dst3, ew3, zn)
    degT = degP.T

    h1 = _tc1(x, W1, degT)
    p1 = _edge(h1, src3, dst3, ew16, znd)
    h2 = _tc2(p1, h1, degT,
              b1.reshape(1, D), bn_gamma.reshape(1, D), bn_beta.reshape(1, D),
              bn_mean.reshape(1, D), bn_var.reshape(1, D), W2)
    p2 = _edge(h2, src3, dst3, ew16, znd)
    return _tc3(p2, h2, degT, b2.reshape(1, D))


# speedup vs baseline: 7.8103x; 7.8103x over previous
import functools

import jax
import jax.numpy as jnp
from jax import lax
from jax.experimental import pallas as pl
from jax.experimental.pallas import tpu as pltpu
from jax.experimental.pallas import tpu_sc as plsc

N = 10000
E = 320000
D = 128
H = D // 2

NC = 2
NS = 16
NW = NC * NS
C = 80

DPW = E // NW
DCH = DPW // C
EPS = E // NS
ECH = EPS // C

SL = 624
TAIL = N - NS * SL

_MESH = plsc.VectorSubcoreMesh(
    core_axis_name="c", subcore_axis_name="s", num_cores=NC, num_subcores=NS
)


@functools.partial(
    pl.kernel,
    out_type=jax.ShapeDtypeStruct((NC, N), jnp.float32),
    mesh=_MESH,
    scratch_types=[
        pltpu.VMEM((DCH, C), jnp.int32),
        pltpu.VMEM((DCH, C), jnp.float32),
        pltpu.VMEM_SHARED((N,), jnp.float32),
    ],
)
def _deg(dst_hbm, ew_hbm, zn_hbm, out_hbm, dstv, ewv, degsh):
    ci = lax.axis_index("c")
    si = lax.axis_index("s")
    wid = ci * NS + si
    pltpu.sync_copy(dst_hbm.at[wid], dstv)
    pltpu.sync_copy(ew_hbm.at[wid], ewv)

    @pl.when(si == 0)
    def _():
        pltpu.sync_copy(zn_hbm, degsh)

    plsc.subcore_barrier()

    def body(j, carry):
        pltpu.sync_copy(ewv.at[j], degsh.at[dstv.at[j]], add=True)
        return carry

    lax.fori_loop(0, DCH, body, 0)
    plsc.subcore_barrier()

    @pl.when(si == 0)
    def _():
        pltpu.sync_copy(degsh, out_hbm.at[ci])


@functools.partial(
    pl.kernel,
    out_type=jax.ShapeDtypeStruct((NC, N, H), jnp.float32),
    mesh=_MESH,
    scratch_types=[
        pltpu.VMEM((ECH, C), jnp.int32),
        pltpu.VMEM((ECH, C), jnp.int32),
        pltpu.VMEM((C, 16), jnp.float32),
        pltpu.VMEM((C, H), jnp.float32),
        pltpu.VMEM_SHARED((N, H), jnp.float32),
        pltpu.SemaphoreType.DMA,
    ],
    compiler_params=pltpu.CompilerParams(use_tc_tiling_on_sc=False),
)
def _edge(h_hbm, src_hbm, dst_hbm, ew16_hbm, znd_hbm, out_hbm,
          srcv, dstv, ew16v, rows, acc, gsem):
    ci = lax.axis_index("c")
    si = lax.axis_index("s")
    pltpu.sync_copy(src_hbm.at[si], srcv)
    pltpu.sync_copy(dst_hbm.at[si], dstv)
    pltpu.sync_copy(znd_hbm.at[pl.ds(si * SL, SL)], acc.at[pl.ds(si * SL, SL)])

    @pl.when(si == 0)
    def _():
        pltpu.sync_copy(znd_hbm.at[pl.ds(NS * SL, TAIL)],
                        acc.at[pl.ds(NS * SL, TAIL)])

    off = (ci * N).astype(jnp.int32)

    def offset_body(j, carry):
        for g in range(C // 16):
            srcv[j, 16 * g:16 * (g + 1)] = srcv[j, 16 * g:16 * (g + 1)] + off
        return carry

    lax.fori_loop(0, ECH, offset_body, 0)
    plsc.subcore_barrier()

    def body(j, carry):
        pltpu.sync_copy(ew16_hbm.at[si, j], ew16v)
        pltpu.async_copy(h_hbm.at[srcv.at[j]], rows, gsem).wait()
        for e in range(C):
            wv = ew16v[e]
            for q in range(H // 16):
                rows[e, 16 * q:16 * (q + 1)] = rows[e, 16 * q:16 * (q + 1)] * wv
        pltpu.sync_copy(rows, acc.at[dstv.at[j]], add=True)
        return carry

    lax.fori_loop(0, ECH, body, 0)
    plsc.subcore_barrier()
    pltpu.sync_copy(acc.at[pl.ds(si * SL, SL)],
                    out_hbm.at[ci, pl.ds(si * SL, SL)])

    @pl.when(si == 0)
    def _():
        pltpu.sync_copy(acc.at[pl.ds(NS * SL, TAIL)],
                        out_hbm.at[ci, pl.ds(NS * SL, TAIL)])


_RB = 2000
_GRID = N // _RB


def _dis_of(degT_blk):
    deg = degT_blk[:, 0:1] + degT_blk[:, 1:2] + 1.0
    return lax.rsqrt(deg)


def _split_store(out_ref, val):
    out_ref[0] = val[:, :H]
    out_ref[1] = val[:, H:]


def _cat(p_ref):
    return jnp.concatenate([p_ref[0], p_ref[1]], axis=1)


def _tc1_body(x_ref, w1_ref, degT_ref, out_ref):
    dis = _dis_of(degT_ref[...])
    _split_store(out_ref, jnp.dot(x_ref[...], w1_ref[...],
                                  preferred_element_type=jnp.float32) * dis)


def _tc2_body(p_ref, h_ref, degT_ref, b1_ref, g_ref, be_ref, mu_ref, va_ref,
              w2_ref, out_ref):
    dis = _dis_of(degT_ref[...])
    t = dis * (_cat(p_ref) + _cat(h_ref)) + b1_ref[...]
    inv = lax.rsqrt(va_ref[...] + 1e-5)
    t = (t - mu_ref[...]) * inv * g_ref[...] + be_ref[...]
    t = jnp.maximum(t, 0.0)
    _split_store(out_ref, jnp.dot(t, w2_ref[...],
                                  preferred_element_type=jnp.float32) * dis)


def _tc3_body(p_ref, h_ref, degT_ref, b2_ref, out_ref):
    dis = _dis_of(degT_ref[...])
    out_ref[...] = dis * (_cat(p_ref) + _cat(h_ref)) + b2_ref[...]


_rowspec = pl.BlockSpec((_RB, D), lambda i: (i, 0))
_fullmat = pl.BlockSpec((D, D), lambda i: (0, 0))
_degspec = pl.BlockSpec((_RB, NC), lambda i: (i, 0))
_vecspec = pl.BlockSpec((1, D), lambda i: (0, 0))
_halfspec = pl.BlockSpec((NC, _RB, H), lambda i: (0, i, 0))

_tc1 = pl.pallas_call(
    _tc1_body,
    grid=(_GRID,),
    in_specs=[_rowspec, _fullmat, _degspec],
    out_specs=_halfspec,
    out_shape=jax.ShapeDtypeStruct((NC, N, H), jnp.float32),
)

_tc2 = pl.pallas_call(
    _tc2_body,
    grid=(_GRID,),
    in_specs=[_halfspec, _halfspec, _degspec,
              _vecspec, _vecspec, _vecspec, _vecspec, _vecspec, _fullmat],
    out_specs=_halfspec,
    out_shape=jax.ShapeDtypeStruct((NC, N, H), jnp.float32),
)

_tc3 = pl.pallas_call(
    _tc3_body,
    grid=(_GRID,),
    in_specs=[_halfspec, _halfspec, _degspec, _vecspec],
    out_specs=_rowspec,
    out_shape=jax.ShapeDtypeStruct((N, D), jnp.float32),
)


def kernel(x, edge_index, edge_weight, W1, b1, bn_gamma, bn_beta, bn_mean,
           bn_var, W2, b2):
    src3 = edge_index[0].reshape(NS, ECH, C)
    dst3 = edge_index[1].reshape(NS, ECH, C)
    dstd = edge_index[1].reshape(NW, DCH, C)
    ewd = edge_weight.reshape(NW, DCH, C)
    ew16 = jnp.broadcast_to(edge_weight[:, None], (E, 16)).reshape(NS, ECH, C, 16)
    zn = jnp.zeros((N,), jnp.float32)
    znd = jnp.zeros((N, H), jnp.float32)

    degP = _deg(dstd, ewd, zn)
    degT = degP.T

    h1 = _tc1(x, W1, degT)
    p1 = _edge(h1.reshape(NC * N, H), src3, dst3, ew16, znd)
    h2 = _tc2(p1, h1, degT,
              b1.reshape(1, D), bn_gamma.reshape(1, D), bn_beta.reshape(1, D),
              bn_mean.reshape(1, D), bn_var.reshape(1, D), W2)
    p2 = _edge(h2.reshape(NC * N, H), src3, dst3, ew16, znd)
    return _tc3(p2, h2, degT, b2.reshape(1, D))


# speedup vs baseline: 17.2882x; 2.2135x over previous
import functools

import jax
import jax.numpy as jnp
from jax import lax
from jax.experimental import pallas as pl
from jax.experimental.pallas import tpu as pltpu
from jax.experimental.pallas import tpu_sc as plsc

N = 10000
E = 320000
D = 128
H = D // 2

NC = 2
NS = 16
NW = NC * NS
C = 80

DPW = E // NW
DCH = DPW // C
EPS = E // NS
ECH = EPS // C

SL = 624
TAIL = N - NS * SL

_MESH = plsc.VectorSubcoreMesh(
    core_axis_name="c", subcore_axis_name="s", num_cores=NC, num_subcores=NS
)


@functools.partial(
    pl.kernel,
    out_type=jax.ShapeDtypeStruct((NC, N), jnp.float32),
    mesh=_MESH,
    scratch_types=[
        pltpu.VMEM((DCH, C), jnp.int32),
        pltpu.VMEM((DCH, C), jnp.float32),
        pltpu.VMEM_SHARED((N,), jnp.float32),
    ],
)
def _deg(dst_hbm, ew_hbm, zn_hbm, out_hbm, dstv, ewv, degsh):
    ci = lax.axis_index("c")
    si = lax.axis_index("s")
    wid = ci * NS + si
    pltpu.sync_copy(dst_hbm.at[wid], dstv)
    pltpu.sync_copy(ew_hbm.at[wid], ewv)

    @pl.when(si == 0)
    def _():
        pltpu.sync_copy(zn_hbm, degsh)

    plsc.subcore_barrier()

    def body(j, carry):
        pltpu.sync_copy(ewv.at[j], degsh.at[dstv.at[j]], add=True)
        return carry

    lax.fori_loop(0, DCH, body, 0)
    plsc.subcore_barrier()

    @pl.when(si == 0)
    def _():
        pltpu.sync_copy(degsh, out_hbm.at[ci])


@functools.partial(
    pl.kernel,
    out_type=jax.ShapeDtypeStruct((NC, N, H), jnp.float32),
    mesh=_MESH,
    scratch_types=[
        pltpu.VMEM((ECH, C), jnp.int32),
        pltpu.VMEM((ECH, C), jnp.int32),
        pltpu.VMEM((ECH, C), jnp.float32),
        pltpu.VMEM((2, C, H), jnp.float32),
        pltpu.VMEM_SHARED((N, H), jnp.float32),
        pltpu.SemaphoreType.DMA,
        pltpu.SemaphoreType.DMA,
    ],
    compiler_params=pltpu.CompilerParams(use_tc_tiling_on_sc=False),
)
def _edge(h_hbm, src_hbm, dst_hbm, ew_hbm, znd_hbm, out_hbm,
          srcv, dstv, eww, rows, acc, gsem, ssem):
    ci = lax.axis_index("c")
    si = lax.axis_index("s")
    pltpu.sync_copy(src_hbm.at[si], srcv)
    pltpu.sync_copy(dst_hbm.at[si], dstv)
    pltpu.sync_copy(ew_hbm.at[si], eww)
    pltpu.sync_copy(znd_hbm.at[pl.ds(si * SL, SL)], acc.at[pl.ds(si * SL, SL)])

    @pl.when(si == 0)
    def _():
        pltpu.sync_copy(znd_hbm.at[pl.ds(NS * SL, TAIL)],
                        acc.at[pl.ds(NS * SL, TAIL)])

    off = (ci * N).astype(jnp.int32)

    def offset_body(j, carry):
        for g in range(C // 16):
            srcv[j, 16 * g:16 * (g + 1)] = srcv[j, 16 * g:16 * (g + 1)] + off
        return carry

    lax.fori_loop(0, ECH, offset_body, 0)
    plsc.subcore_barrier()

    pltpu.async_copy(h_hbm.at[srcv.at[0]], rows.at[0], gsem)

    def chunk(j, b):
        pltpu.make_async_copy(h_hbm.at[srcv.at[j]], rows.at[b], gsem).wait()

        @pl.when(j > 0)
        def _():
            pltpu.make_async_copy(rows.at[1 - b],
                                  acc.at[dstv.at[j - 1]], ssem).wait()

        @pl.when(j + 1 < ECH)
        def _():
            pltpu.async_copy(h_hbm.at[srcv.at[j + 1]], rows.at[1 - b], gsem)

        for g in range(C // 16):
            ewg = eww[j, 16 * g:16 * (g + 1)]
            for l in range(16):
                wv = lax.gather(
                    ewg, jnp.full((16, 1), l, jnp.int32),
                    lax.GatherDimensionNumbers(offset_dims=(),
                                               collapsed_slice_dims=(0,),
                                               start_index_map=(0,)),
                    slice_sizes=(1,),
                    mode=lax.GatherScatterMode.PROMISE_IN_BOUNDS)
                e = 16 * g + l
                for q in range(H // 16):
                    rows[b, e, 16 * q:16 * (q + 1)] = (
                        rows[b, e, 16 * q:16 * (q + 1)] * wv)
        pltpu.async_copy(rows.at[b], acc.at[dstv.at[j]], ssem, add=True)

    def outer(jo, carry):
        chunk(2 * jo, 0)
        chunk(2 * jo + 1, 1)
        return carry

    lax.fori_loop(0, ECH // 2, outer, 0)
    pltpu.make_async_copy(rows.at[1], acc.at[dstv.at[ECH - 1]], ssem).wait()
    plsc.subcore_barrier()
    pltpu.sync_copy(acc.at[pl.ds(si * SL, SL)],
                    out_hbm.at[ci, pl.ds(si * SL, SL)])

    @pl.when(si == 0)
    def _():
        pltpu.sync_copy(acc.at[pl.ds(NS * SL, TAIL)],
                        out_hbm.at[ci, pl.ds(NS * SL, TAIL)])


_RB = 2000
_GRID = N // _RB


def _dis_of(degT_blk):
    deg = degT_blk[:, 0:1] + degT_blk[:, 1:2] + 1.0
    return lax.rsqrt(deg)


def _split_store(out_ref, val):
    out_ref[0] = val[:, :H]
    out_ref[1] = val[:, H:]


def _cat(p_ref):
    return jnp.concatenate([p_ref[0], p_ref[1]], axis=1)


def _tc1_body(x_ref, w1_ref, degT_ref, out_ref):
    dis = _dis_of(degT_ref[...])
    _split_store(out_ref, jnp.dot(x_ref[...], w1_ref[...],
                                  preferred_element_type=jnp.float32) * dis)


def _tc2_body(p_ref, h_ref, degT_ref, b1_ref, g_ref, be_ref, mu_ref, va_ref,
              w2_ref, out_ref):
    dis = _dis_of(degT_ref[...])
    t = dis * (_cat(p_ref) + _cat(h_ref)) + b1_ref[...]
    inv = lax.rsqrt(va_ref[...] + 1e-5)
    t = (t - mu_ref[...]) * inv * g_ref[...] + be_ref[...]
    t = jnp.maximum(t, 0.0)
    _split_store(out_ref, jnp.dot(t, w2_ref[...],
                                  preferred_element_type=jnp.float32) * dis)


def _tc3_body(p_ref, h_ref, degT_ref, b2_ref, out_ref):
    dis = _dis_of(degT_ref[...])
    out_ref[...] = dis * (_cat(p_ref) + _cat(h_ref)) + b2_ref[...]


_rowspec = pl.BlockSpec((_RB, D), lambda i: (i, 0))
_fullmat = pl.BlockSpec((D, D), lambda i: (0, 0))
_degspec = pl.BlockSpec((_RB, NC), lambda i: (i, 0))
_vecspec = pl.BlockSpec((1, D), lambda i: (0, 0))
_halfspec = pl.BlockSpec((NC, _RB, H), lambda i: (0, i, 0))

_tc1 = pl.pallas_call(
    _tc1_body,
    grid=(_GRID,),
    in_specs=[_rowspec, _fullmat, _degspec],
    out_specs=_halfspec,
    out_shape=jax.ShapeDtypeStruct((NC, N, H), jnp.float32),
)

_tc2 = pl.pallas_call(
    _tc2_body,
    grid=(_GRID,),
    in_specs=[_halfspec, _halfspec, _degspec,
              _vecspec, _vecspec, _vecspec, _vecspec, _vecspec, _fullmat],
    out_specs=_halfspec,
    out_shape=jax.ShapeDtypeStruct((NC, N, H), jnp.float32),
)

_tc3 = pl.pallas_call(
    _tc3_body,
    grid=(_GRID,),
    in_specs=[_halfspec, _halfspec, _degspec, _vecspec],
    out_specs=_rowspec,
    out_shape=jax.ShapeDtypeStruct((N, D), jnp.float32),
)


def kernel(x, edge_index, edge_weight, W1, b1, bn_gamma, bn_beta, bn_mean,
           bn_var, W2, b2):
    src3 = edge_index[0].reshape(NS, ECH, C)
    dst3 = edge_index[1].reshape(NS, ECH, C)
    dstd = edge_index[1].reshape(NW, DCH, C)
    ewd = edge_weight.reshape(NW, DCH, C)
    ew3 = edge_weight.reshape(NS, ECH, C)
    zn = jnp.zeros((N,), jnp.float32)
    znd = jnp.zeros((N, H), jnp.float32)

    degP = _deg(dstd, ewd, zn)
    degT = degP.T

    h1 = _tc1(x, W1, degT)
    p1 = _edge(h1.reshape(NC * N, H), src3, dst3, ew3, znd)
    h2 = _tc2(p1, h1, degT,
              b1.reshape(1, D), bn_gamma.reshape(1, D), bn_beta.reshape(1, D),
              bn_mean.reshape(1, D), bn_var.reshape(1, D), W2)
    p2 = _edge(h2.reshape(NC * N, H), src3, dst3, ew3, znd)
    return _tc3(p2, h2, degT, b2.reshape(1, D))


# speedup vs baseline: 23.0262x; 1.3319x over previous
import functools

import jax
import jax.numpy as jnp
from jax import lax
from jax.experimental import pallas as pl
from jax.experimental.pallas import tpu as pltpu
from jax.experimental.pallas import tpu_sc as plsc

N = 10000
E = 320000
D = 128
H = D // 2

NC = 2
NS = 16
NW = NC * NS
C = 80

DPW = E // NW
DCH = DPW // C
EPS = E // NS
ECH = EPS // C

SL = 624
TAIL = N - NS * SL

_MESH = plsc.VectorSubcoreMesh(
    core_axis_name="c", subcore_axis_name="s", num_cores=NC, num_subcores=NS
)


@functools.partial(
    pl.kernel,
    out_type=jax.ShapeDtypeStruct((NC, N), jnp.float32),
    mesh=_MESH,
    scratch_types=[
        pltpu.VMEM((DCH, C), jnp.int32),
        pltpu.VMEM((DCH, C), jnp.float32),
        pltpu.VMEM_SHARED((N,), jnp.float32),
    ],
)
def _deg(dst_hbm, ew_hbm, zn_hbm, out_hbm, dstv, ewv, degsh):
    ci = lax.axis_index("c")
    si = lax.axis_index("s")
    wid = ci * NS + si
    pltpu.sync_copy(dst_hbm.at[wid], dstv)
    pltpu.sync_copy(ew_hbm.at[wid], ewv)

    @pl.when(si == 0)
    def _():
        pltpu.sync_copy(zn_hbm, degsh)

    plsc.subcore_barrier()

    def body(j, carry):
        pltpu.sync_copy(ewv.at[j], degsh.at[dstv.at[j]], add=True)
        return carry

    lax.fori_loop(0, DCH, body, 0)
    plsc.subcore_barrier()

    @pl.when(si == 0)
    def _():
        pltpu.sync_copy(degsh, out_hbm.at[ci])


@functools.partial(
    pl.kernel,
    out_type=jax.ShapeDtypeStruct((NC, N, H), jnp.float32),
    mesh=_MESH,
    scratch_types=[
        pltpu.VMEM((ECH, C), jnp.int32),
        pltpu.VMEM((ECH, C), jnp.int32),
        pltpu.VMEM((ECH, C), jnp.float32),
        pltpu.VMEM((5, C, H), jnp.float32),
        pltpu.VMEM_SHARED((N, H), jnp.float32),
        pltpu.SemaphoreType.DMA,
        pltpu.SemaphoreType.DMA,
    ],
    compiler_params=pltpu.CompilerParams(use_tc_tiling_on_sc=False),
)
def _edge(h_hbm, src_hbm, dst_hbm, ew_hbm, znd_hbm, out_hbm,
          srcv, dstv, eww, rows, acc, gsem, ssem):
    ci = lax.axis_index("c")
    si = lax.axis_index("s")
    pltpu.sync_copy(src_hbm.at[si], srcv)
    pltpu.sync_copy(dst_hbm.at[si], dstv)
    pltpu.sync_copy(ew_hbm.at[si], eww)
    pltpu.sync_copy(znd_hbm.at[pl.ds(si * SL, SL)], acc.at[pl.ds(si * SL, SL)])

    @pl.when(si == 0)
    def _():
        pltpu.sync_copy(znd_hbm.at[pl.ds(NS * SL, TAIL)],
                        acc.at[pl.ds(NS * SL, TAIL)])

    off = (ci * N).astype(jnp.int32)

    def offset_body(j, carry):
        for g in range(C // 16):
            srcv[j, 16 * g:16 * (g + 1)] = srcv[j, 16 * g:16 * (g + 1)] + off
        return carry

    lax.fori_loop(0, ECH, offset_body, 0)
    plsc.subcore_barrier()

    NB = 5
    PF = 3
    for p in range(PF):
        pltpu.async_copy(h_hbm.at[srcv.at[p]], rows.at[p], gsem)

    def chunk(j, b):
        pltpu.make_async_copy(h_hbm.at[srcv.at[j]], rows.at[b], gsem).wait()

        @pl.when(j >= NB - PF)
        def _():
            pltpu.make_async_copy(rows.at[(b + PF) % NB],
                                  acc.at[dstv.at[j - (NB - PF)]], ssem).wait()

        @pl.when(j + PF < ECH)
        def _():
            pltpu.async_copy(h_hbm.at[srcv.at[j + PF]],
                             rows.at[(b + PF) % NB], gsem)

        for g in range(C // 16):
            ewg = eww[j, 16 * g:16 * (g + 1)]
            for l in range(16):
                wv = lax.gather(
                    ewg, jnp.full((16, 1), l, jnp.int32),
                    lax.GatherDimensionNumbers(offset_dims=(),
                                               collapsed_slice_dims=(0,),
                                               start_index_map=(0,)),
                    slice_sizes=(1,),
                    mode=lax.GatherScatterMode.PROMISE_IN_BOUNDS)
                e = 16 * g + l
                for q in range(H // 16):
                    rows[b, e, 16 * q:16 * (q + 1)] = (
                        rows[b, e, 16 * q:16 * (q + 1)] * wv)
        pltpu.async_copy(rows.at[b], acc.at[dstv.at[j]], ssem, add=True)

    def outer(jo, carry):
        for b in range(NB):
            chunk(NB * jo + b, b)
        return carry

    lax.fori_loop(0, ECH // NB, outer, 0)
    for t in range(NB - PF):
        pltpu.make_async_copy(rows.at[(ECH - 1 - t) % NB],
                              acc.at[dstv.at[ECH - 1 - t]], ssem).wait()
    plsc.subcore_barrier()
    pltpu.sync_copy(acc.at[pl.ds(si * SL, SL)],
                    out_hbm.at[ci, pl.ds(si * SL, SL)])

    @pl.when(si == 0)
    def _():
        pltpu.sync_copy(acc.at[pl.ds(NS * SL, TAIL)],
                        out_hbm.at[ci, pl.ds(NS * SL, TAIL)])


_RB = 2000
_GRID = N // _RB


def _dis_of(degT_blk):
    deg = degT_blk[:, 0:1] + degT_blk[:, 1:2] + 1.0
    return lax.rsqrt(deg)


def _split_store(out_ref, val):
    out_ref[0] = val[:, :H]
    out_ref[1] = val[:, H:]


def _cat(p_ref):
    return jnp.concatenate([p_ref[0], p_ref[1]], axis=1)


def _tc1_body(x_ref, w1_ref, degT_ref, out_ref):
    dis = _dis_of(degT_ref[...])
    _split_store(out_ref, jnp.dot(x_ref[...], w1_ref[...],
                                  preferred_element_type=jnp.float32) * dis)


def _tc2_body(p_ref, h_ref, degT_ref, b1_ref, g_ref, be_ref, mu_ref, va_ref,
              w2_ref, out_ref):
    dis = _dis_of(degT_ref[...])
    t = dis * (_cat(p_ref) + _cat(h_ref)) + b1_ref[...]
    inv = lax.rsqrt(va_ref[...] + 1e-5)
    t = (t - mu_ref[...]) * inv * g_ref[...] + be_ref[...]
    t = jnp.maximum(t, 0.0)
    _split_store(out_ref, jnp.dot(t, w2_ref[...],
                                  preferred_element_type=jnp.float32) * dis)


def _tc3_body(p_ref, h_ref, degT_ref, b2_ref, out_ref):
    dis = _dis_of(degT_ref[...])
    out_ref[...] = dis * (_cat(p_ref) + _cat(h_ref)) + b2_ref[...]


_rowspec = pl.BlockSpec((_RB, D), lambda i: (i, 0))
_fullmat = pl.BlockSpec((D, D), lambda i: (0, 0))
_degspec = pl.BlockSpec((_RB, NC), lambda i: (i, 0))
_vecspec = pl.BlockSpec((1, D), lambda i: (0, 0))
_halfspec = pl.BlockSpec((NC, _RB, H), lambda i: (0, i, 0))

_tc1 = pl.pallas_call(
    _tc1_body,
    grid=(_GRID,),
    in_specs=[_rowspec, _fullmat, _degspec],
    out_specs=_halfspec,
    out_shape=jax.ShapeDtypeStruct((NC, N, H), jnp.float32),
)

_tc2 = pl.pallas_call(
    _tc2_body,
    grid=(_GRID,),
    in_specs=[_halfspec, _halfspec, _degspec,
              _vecspec, _vecspec, _vecspec, _vecspec, _vecspec, _fullmat],
    out_specs=_halfspec,
    out_shape=jax.ShapeDtypeStruct((NC, N, H), jnp.float32),
)

_tc3 = pl.pallas_call(
    _tc3_body,
    grid=(_GRID,),
    in_specs=[_halfspec, _halfspec, _degspec, _vecspec],
    out_specs=_rowspec,
    out_shape=jax.ShapeDtypeStruct((N, D), jnp.float32),
)


def kernel(x, edge_index, edge_weight, W1, b1, bn_gamma, bn_beta, bn_mean,
           bn_var, W2, b2):
    src3 = edge_index[0].reshape(NS, ECH, C)
    dst3 = edge_index[1].reshape(NS, ECH, C)
    dstd = edge_index[1].reshape(NW, DCH, C)
    ewd = edge_weight.reshape(NW, DCH, C)
    ew3 = edge_weight.reshape(NS, ECH, C)
    zn = jnp.zeros((N,), jnp.float32)
    znd = jnp.zeros((N, H), jnp.float32)

    degP = _deg(dstd, ewd, zn)
    degT = degP.T

    h1 = _tc1(x, W1, degT)
    p1 = _edge(h1.reshape(NC * N, H), src3, dst3, ew3, znd)
    h2 = _tc2(p1, h1, degT,
              b1.reshape(1, D), bn_gamma.reshape(1, D), bn_beta.reshape(1, D),
              bn_mean.reshape(1, D), bn_var.reshape(1, D), W2)
    p2 = _edge(h2.reshape(NC * N, H), src3, dst3, ew3, znd)
    return _tc3(p2, h2, degT, b2.reshape(1, D))
